# Initial kernel scaffold; baseline (speedup 1.0000x reference)
import jax
import jax.numpy as jnp
from jax.experimental import pallas as pl


def kernel(rna_norm, rna_counts, rna_libsize, cas_norm, cas_counts, cas_libsize, adt_norm, edge_index, W1, b1, a1s, a1d, W2, b2, a2s, a2d, W3, b3, Wd1, bd1, Wd2, bd2, Wpi, bpi, Wdisp, bdisp, Wmean, bmean, Wrec, brec):
    raise NotImplementedError("write your pallas kernel here")


# speedup vs baseline: 4.5931x; 4.5931x over previous
import functools

import jax
import jax.numpy as jnp
from jax import lax
from jax.experimental import pallas as pl
from jax.experimental.pallas import tpu as pltpu
from jax.experimental.pallas import tpu_sc as plsc

N = 10000
E = 320000
RNA_DIM = 128
D1, D2 = 256, 128
D_LAT = 50

NCORES = 2
NSUB = 16
NW = NCORES * NSUB
CHUNK = 128
NP = 10240
E_PAD = ((E + NW * CHUNK - 1) // (NW * CHUNK)) * (NW * CHUNK)
EPT32 = E_PAD // NW
CH32 = EPT32 // CHUNK
EPT16 = E_PAD // NSUB
CH16 = EPT16 // CHUNK

_HI = lax.Precision.HIGHEST


def _elu(x):
    return jnp.where(x > 0, x, jnp.exp(jnp.minimum(x, 0.0)) - 1.0)


def _tc1_body(x_ref, w_ref, b_ref, as_ref, ad_ref, tab_ref, ss_ref, sd_ref):
    h = jnp.dot(x_ref[...], w_ref[...], precision=_HI,
                preferred_element_type=jnp.float32) + b_ref[...]
    tab_ref[0, :, :] = h[:, :D1 // 2]
    tab_ref[1, :, :] = h[:, D1 // 2:]
    ss_ref[...] = jnp.dot(h, as_ref[...], precision=_HI)
    sd_ref[...] = jnp.dot(h, ad_ref[...], precision=_HI)


def _tc1(x, W1, b1, a1s, a1d):
    bn = 1000
    grid = (N // bn,)
    return pl.pallas_call(
        _tc1_body,
        grid=grid,
        in_specs=[
            pl.BlockSpec((bn, RNA_DIM), lambda i: (i, 0)),
            pl.BlockSpec((RNA_DIM, D1), lambda i: (0, 0)),
            pl.BlockSpec((1, D1), lambda i: (0, 0)),
            pl.BlockSpec((D1, 1), lambda i: (0, 0)),
            pl.BlockSpec((D1, 1), lambda i: (0, 0)),
        ],
        out_specs=[
            pl.BlockSpec((2, bn, D1 // 2), lambda i: (0, i, 0)),
            pl.BlockSpec((bn, 1), lambda i: (i, 0)),
            pl.BlockSpec((bn, 1), lambda i: (i, 0)),
        ],
        out_shape=[
            jax.ShapeDtypeStruct((2, N, D1 // 2), jnp.float32),
            jax.ShapeDtypeStruct((N, 1), jnp.float32),
            jax.ShapeDtypeStruct((N, 1), jnp.float32),
        ],
    )(x, W1, b1.reshape(1, D1), a1s.reshape(D1, 1), a1d.reshape(D1, 1))


def _tc2_body(a0_ref, a1_ref, w_ref, b_ref, as_ref, ad_ref,
              tab_ref, ss_ref, sd_ref):
    dh = D1 // 2
    inv = 1.0 / (a0_ref[:, dh:dh + 1] + 1e-16)
    hl = _elu(a0_ref[:, :dh] * inv)
    hr = _elu(a1_ref[:, :dh] * inv)
    h = (jnp.dot(hl, w_ref[:dh, :], precision=_HI,
                 preferred_element_type=jnp.float32)
         + jnp.dot(hr, w_ref[dh:, :], precision=_HI,
                   preferred_element_type=jnp.float32)
         + b_ref[...])
    tab_ref[0, :, :] = h[:, :D2 // 2]
    tab_ref[1, :, :] = h[:, D2 // 2:]
    ss_ref[...] = jnp.dot(h, as_ref[...], precision=_HI)
    sd_ref[...] = jnp.dot(h, ad_ref[...], precision=_HI)


def _tc2(acc0, acc1, W2, b2, a2s, a2d):
    bn = 1000
    grid = (N // bn,)
    wa = D1 // 2 + 16
    return pl.pallas_call(
        _tc2_body,
        grid=grid,
        in_specs=[
            pl.BlockSpec((bn, wa), lambda i: (i, 0)),
            pl.BlockSpec((bn, wa), lambda i: (i, 0)),
            pl.BlockSpec((D1, D2), lambda i: (0, 0)),
            pl.BlockSpec((1, D2), lambda i: (0, 0)),
            pl.BlockSpec((D2, 1), lambda i: (0, 0)),
            pl.BlockSpec((D2, 1), lambda i: (0, 0)),
        ],
        out_specs=[
            pl.BlockSpec((2, bn, D2 // 2), lambda i: (0, i, 0)),
            pl.BlockSpec((bn, 1), lambda i: (i, 0)),
            pl.BlockSpec((bn, 1), lambda i: (i, 0)),
        ],
        out_shape=[
            jax.ShapeDtypeStruct((2, N, D2 // 2), jnp.float32),
            jax.ShapeDtypeStruct((N, 1), jnp.float32),
            jax.ShapeDtypeStruct((N, 1), jnp.float32),
        ],
    )(acc0, acc1, W2, b2.reshape(1, D2), a2s.reshape(D2, 1),
      a2d.reshape(D2, 1))


_LANCZOS = (676.5203681218851, -1259.1392167224028, 771.3234287776531,
            -176.6150291621406, 12.507343278686905, -0.13857109526572012,
            9.984369578019572e-6, 1.5056327351493116e-7)


def _lgamma(a):
    t = a + 7.5
    s = jnp.float32(0.99999999999980993)
    for i, c in enumerate(_LANCZOS):
        s = s + jnp.float32(c) / (a + jnp.float32(i + 1))
    return (jnp.float32(0.9189385332046727) + (a + 0.5) * jnp.log(t) - t
            + jnp.log(s) - jnp.log(a))


def _softplus(x):
    return jnp.maximum(x, 0.0) + jnp.log1p(jnp.exp(-jnp.abs(x)))


def _tc3_body(a0_ref, a1_ref, xn_ref, xc_ref, lib_ref,
              w3_ref, b3_ref, wd1_ref, bd1_ref, wd2_ref, bd2_ref,
              wpi_ref, bpi_ref, wdisp_ref, bdisp_ref,
              wmean_ref, bmean_ref, wrec_ref, brec_ref,
              out_ref):
    i = pl.program_id(0)
    dh = D2 // 2
    inv = 1.0 / (a0_ref[:, dh:dh + 1] + 1e-16)
    h = jnp.concatenate(
        [_elu(a0_ref[:, :dh] * inv), _elu(a1_ref[:, :dh] * inv)], axis=1)
    z = jnp.dot(h, w3_ref[...], precision=_HI,
                preferred_element_type=jnp.float32) + b3_ref[...]
    d = jnp.maximum(jnp.dot(z, wd1_ref[...], precision=_HI,
                            preferred_element_type=jnp.float32)
                    + bd1_ref[...], 0.0)
    d = jnp.maximum(jnp.dot(d, wd2_ref[...], precision=_HI,
                            preferred_element_type=jnp.float32)
                    + bd2_ref[...], 0.0)
    pi = jax.nn.sigmoid(jnp.dot(d, wpi_ref[...], precision=_HI,
                                preferred_element_type=jnp.float32)
                        + bpi_ref[...])
    disp = jnp.clip(
        _softplus(jnp.dot(d, wdisp_ref[...], precision=_HI,
                          preferred_element_type=jnp.float32)
                  + bdisp_ref[...]), 1e-4, 1e4)
    mean = jnp.clip(
        jnp.exp(jnp.clip(jnp.dot(d, wmean_ref[...], precision=_HI,
                                 preferred_element_type=jnp.float32)
                         + bmean_ref[...], -15.0, 15.0)), 1e-5, 1e6)
    recons = jnp.dot(d, wrec_ref[...], precision=_HI,
                     preferred_element_type=jnp.float32) + brec_ref[...]

    eps = 1e-10
    x = xc_ref[...]
    mu = mean * lib_ref[...]
    t1 = _lgamma(disp + eps) + _lgamma(x + 1.0) - _lgamma(x + disp + eps)
    t2 = ((disp + x) * jnp.log(1.0 + mu / (disp + eps))
          + x * (jnp.log(disp + eps) - jnp.log(mu + eps)))
    nb_nll = t1 + t2
    nb_case = nb_nll - jnp.log(1.0 - pi + eps)
    ratio = disp / (disp + mu + eps)
    zero_nb = jnp.exp(disp * jnp.log(ratio))
    zero_case = -jnp.log(pi + (1.0 - pi) * zero_nb + eps)
    res = jnp.where(x < 1e-8, zero_case, nb_case) + 0.5 * jnp.square(pi)

    nll_sum = jnp.sum(res)
    mse_sum = jnp.sum(jnp.square(recons - xn_ref[...]))

    @pl.when(i == 0)
    def _():
        out_ref[0] = 0.0
        out_ref[1] = 0.0

    out_ref[0] += nll_sum
    out_ref[1] += mse_sum

    @pl.when(i == pl.num_programs(0) - 1)
    def _():
        cnt = jnp.float32(N * RNA_DIM)
        out_ref[0] = out_ref[0] / cnt
        out_ref[1] = out_ref[1] / cnt


def _tc3(acc0, acc1, rna_norm, rna_counts, rna_libsize,
         W3, b3, Wd1, bd1, Wd2, bd2, Wpi, bpi, Wdisp, bdisp,
         Wmean, bmean, Wrec, brec):
    bn = 1000
    grid = (N // bn,)
    wa = D2 // 2 + 16

    def full(shape):
        return pl.BlockSpec(shape, lambda i: tuple(0 for _ in shape))

    return pl.pallas_call(
        _tc3_body,
        grid=grid,
        in_specs=[
            pl.BlockSpec((bn, wa), lambda i: (i, 0)),
            pl.BlockSpec((bn, wa), lambda i: (i, 0)),
            pl.BlockSpec((bn, RNA_DIM), lambda i: (i, 0)),
            pl.BlockSpec((bn, RNA_DIM), lambda i: (i, 0)),
            pl.BlockSpec((bn, 1), lambda i: (i, 0)),
            full((D2, D_LAT)), full((1, D_LAT)),
            full((D_LAT, D2)), full((1, D2)),
            full((D2, D1)), full((1, D1)),
            full((D1, RNA_DIM)), full((1, RNA_DIM)),
            full((D1, RNA_DIM)), full((1, RNA_DIM)),
            full((D1, RNA_DIM)), full((1, RNA_DIM)),
            full((D1, RNA_DIM)), full((1, RNA_DIM)),
        ],
        out_specs=pl.BlockSpec(memory_space=pltpu.SMEM),
        out_shape=jax.ShapeDtypeStruct((2,), jnp.float32),
    )(acc0, acc1, rna_norm, rna_counts, rna_libsize,
      W3, b3.reshape(1, D_LAT), Wd1, bd1.reshape(1, D2),
      Wd2, bd2.reshape(1, D1), Wpi, bpi.reshape(1, RNA_DIM),
      Wdisp, bdisp.reshape(1, RNA_DIM), Wmean, bmean.reshape(1, RNA_DIM),
      Wrec, brec.reshape(1, RNA_DIM))


def _sc_mesh():
    return plsc.VectorSubcoreMesh(core_axis_name="c", subcore_axis_name="s",
                                  num_cores=NCORES, num_subcores=NSUB)


def _sc_compiler_params():
    return pltpu.CompilerParams(needs_layout_passes=False,
                                use_tc_tiling_on_sc=False)


def _make_sc_score():
    def body(ss_hbm, sd_hbm, src_hbm, dst_hbm, w_hbm,
             ss_v, sd_v, src_v, dst_v, w_out_v, red_v):
        cid = lax.axis_index("c")
        sid = lax.axis_index("s")
        wid = sid * NCORES + cid

        pltpu.sync_copy(ss_hbm, ss_v)
        pltpu.sync_copy(sd_hbm, sd_v)

        def mx_body(i, carry):
            ms, md = carry
            ms = jnp.maximum(ms, ss_v[pl.ds(i * 16, 16)])
            md = jnp.maximum(md, sd_v[pl.ds(i * 16, 16)])
            return ms, md

        ninit = jnp.full((16,), -jnp.inf, jnp.float32)
        ms, md = lax.fori_loop(0, N // 16, mx_body, (ninit, ninit))

        lane = lax.iota(jnp.int32, 16)

        def _allmax(vec):
            red_v[...] = vec
            for s in (1, 2, 4, 8):
                cur = red_v[...]
                shuf = plsc.load_gather(red_v, [lane ^ s])
                red_v[...] = jnp.maximum(cur, shuf)
            return red_v[...]

        msum = _allmax(ms) + _allmax(md)
        m_vec = jnp.maximum(msum, 0.2 * msum)

        tile_start = wid * EPT32

        def chunk_body(c, _):
            base = tile_start + c * CHUNK
            pltpu.sync_copy(src_hbm.at[pl.ds(base, CHUNK)], src_v)
            pltpu.sync_copy(dst_hbm.at[pl.ds(base, CHUNK)], dst_v)
            for i in range(CHUNK // 16):
                sv = src_v[pl.ds(i * 16, 16)]
                dv = dst_v[pl.ds(i * 16, 16)]
                e = plsc.load_gather(ss_v, [sv]) + plsc.load_gather(sd_v, [dv])
                e = jnp.maximum(e, 0.2 * e)
                w = jnp.exp(e - m_vec)
                gid = jnp.full((16,), base + i * 16, jnp.int32) + lane
                w_out_v[pl.ds(i * 16, 16)] = jnp.where(gid < E, w, 0.0)
            pltpu.sync_copy(w_out_v, w_hbm.at[pl.ds(base, CHUNK)])
            return 0

        lax.fori_loop(0, CH32, chunk_body, 0)

    return pl.kernel(
        body,
        out_type=jax.ShapeDtypeStruct((E_PAD,), jnp.float32),
        mesh=_sc_mesh(),
        compiler_params=_sc_compiler_params(),
        scratch_types=[
            pltpu.VMEM((N,), jnp.float32),
            pltpu.VMEM((N,), jnp.float32),
            pltpu.VMEM((CHUNK,), jnp.int32),
            pltpu.VMEM((CHUNK,), jnp.int32),
            pltpu.VMEM((CHUNK,), jnp.float32),
            pltpu.VMEM((16,), jnp.float32),
        ],
    )


def _make_sc_rows(dh):
    wa = dh + 16
    nvec = dh // 16
    rpt = NP // NSUB

    def body(tab_hbm, w_hbm, src_hbm, dst_hbm, out_hbm,
             src_v, dst_v, gidx_v, w_v, rows_v, scaled_v, acc_sh, sem):
        cid = lax.axis_index("c")
        sid = lax.axis_index("s")
        core_off = cid * N
        out_off = cid * NP
        lane = lax.iota(jnp.int32, 16)
        lane0 = lane == 0
        zero16 = jnp.zeros((16,), jnp.float32)

        def z_body(j, _):
            for v in range(wa // 16):
                scaled_v[j, pl.ds(v * 16, 16)] = zero16
            return 0

        lax.fori_loop(0, CHUNK, z_body, 0)
        for k in range(rpt // CHUNK):
            pltpu.sync_copy(
                scaled_v, acc_sh.at[pl.ds(sid * rpt + k * CHUNK, CHUNK)])
        plsc.subcore_barrier()

        tile_start = sid * EPT16

        def chunk_body(c, _):
            base = tile_start + c * CHUNK
            pltpu.sync_copy(src_hbm.at[pl.ds(base, CHUNK)], src_v)
            pltpu.sync_copy(dst_hbm.at[pl.ds(base, CHUNK)], dst_v)
            pltpu.sync_copy(w_hbm.at[pl.ds(base, CHUNK)], w_v)
            for i in range(CHUNK // 16):
                gidx_v[pl.ds(i * 16, 16)] = (
                    src_v[pl.ds(i * 16, 16)]
                    + jnp.full((16,), core_off, jnp.int32))
            pltpu.async_copy(tab_hbm.at[gidx_v], rows_v, sem).wait()

            def row_body(j, _):
                wsp = plsc.load_gather(w_v, [jnp.full((16,), j, jnp.int32)])
                for v in range(nvec):
                    scaled_v[j, pl.ds(v * 16, 16)] = (
                        rows_v[j, pl.ds(v * 16, 16)] * wsp)
                scaled_v[j, pl.ds(dh, 16)] = jnp.where(lane0, wsp, 0.0)
                return 0

            lax.fori_loop(0, CHUNK, row_body, 0)
            pltpu.sync_copy(scaled_v, acc_sh.at[dst_v], add=True)
            return 0

        lax.fori_loop(0, CH16, chunk_body, 0)
        plsc.subcore_barrier()
        pltpu.sync_copy(
            acc_sh.at[pl.ds(sid * rpt, rpt)],
            out_hbm.at[pl.ds(out_off + sid * rpt, rpt)])

    return pl.kernel(
        body,
        out_type=jax.ShapeDtypeStruct((2 * NP, wa), jnp.float32),
        mesh=_sc_mesh(),
        compiler_params=_sc_compiler_params(),
        scratch_types=[
            pltpu.VMEM((CHUNK,), jnp.int32),
            pltpu.VMEM((CHUNK,), jnp.int32),
            pltpu.VMEM((CHUNK,), jnp.int32),
            pltpu.VMEM((CHUNK,), jnp.float32),
            pltpu.VMEM((CHUNK, dh), jnp.float32),
            pltpu.VMEM((CHUNK, wa), jnp.float32),
            pltpu.VMEM_SHARED((NP, wa), jnp.float32),
            pltpu.SemaphoreType.DMA,
        ],
    )


_sc_score = functools.lru_cache(maxsize=None)(_make_sc_score)
_sc_rows = functools.lru_cache(maxsize=None)(_make_sc_rows)


def kernel(rna_norm, rna_counts, rna_libsize, cas_norm, cas_counts,
           cas_libsize, adt_norm, edge_index,
           W1, b1, a1s, a1d, W2, b2, a2s, a2d, W3, b3,
           Wd1, bd1, Wd2, bd2, Wpi, bpi, Wdisp, bdisp, Wmean, bmean,
           Wrec, brec):
    src = edge_index[0]
    dst = edge_index[1]
    pad = jnp.zeros((E_PAD - E,), jnp.int32)
    src_p = jnp.concatenate([src, pad])
    dst_p = jnp.concatenate([dst, pad])

    tab1, ss1, sd1 = _tc1(rna_norm, W1, b1, a1s, a1d)
    w1 = _sc_score()(ss1.reshape(N), sd1.reshape(N), src_p, dst_p)
    acc1 = _sc_rows(D1 // 2)(tab1.reshape(2 * N, D1 // 2), w1, src_p, dst_p)
    tab2, ss2, sd2 = _tc2(acc1[:N], acc1[NP:NP + N], W2, b2, a2s, a2d)
    w2 = _sc_score()(ss2.reshape(N), sd2.reshape(N), src_p, dst_p)
    acc2 = _sc_rows(D2 // 2)(tab2.reshape(2 * N, D2 // 2), w2, src_p, dst_p)
    return _tc3(acc2[:N], acc2[NP:NP + N], rna_norm, rna_counts, rna_libsize,
                W3, b3, Wd1, bd1, Wd2, bd2, Wpi, bpi, Wdisp, bdisp,
                Wmean, bmean, Wrec, brec)
